# baseline (speedup 1.0000x reference)
import jax
import jax.numpy as jnp
from jax import lax
from jax.experimental import pallas as pl
from jax.experimental.pallas import tpu as pltpu
from jax.experimental.pallas import tpu_sc as plsc

_N = 10000
_E = 320000
_D = 128
_B = 64

_NC = 2
_NS = 16
_NW = _NC * _NS
_EPT = _E // _NW
_K = 104
_NFULL = _EPT // _K
_REM = _EPT - _NFULL * _K
_NP = 10112
_RPT = _NP // _NS
_ZR = 32
_UN = 4

_HI = 80


def _zero_acc(zb, acc_sh, row0, semz):
    @pl.loop(0, _ZR)
    def _(r):
        @pl.loop(0, _D, step=16)
        def _(cc):
            zb[r, pl.ds(cc, 16)] = jnp.zeros((16,), jnp.float32)

    nfull = _RPT // _ZR
    rem = _RPT - nfull * _ZR
    for i in range(nfull):
        pltpu.async_copy(zb, acc_sh.at[pl.ds(row0 + i * _ZR, _ZR)], semz)
    pltpu.async_copy(zb.at[pl.ds(0, rem)],
                     acc_sh.at[pl.ds(row0 + nfull * _ZR, rem)], semz)
    for i in range(nfull):
        pltpu.make_async_copy(zb, acc_sh.at[pl.ds(row0 + i * _ZR, _ZR)],
                              semz).wait()
    pltpu.make_async_copy(zb.at[pl.ds(0, rem)],
                          acc_sh.at[pl.ds(row0 + nfull * _ZR, rem)],
                          semz).wait()


def _seg_body(p_hbm, src_hbm, dst_hbm, agg_hbm, acc_sh,
              sb0, sb1, sb2, sb3, db0, db1, db2, db3, rb0, rb1, sr, dr, zb,
              semg0, semg1, semi0, semi1, semi2, semi3, sems0, sems1, semz):
    core = lax.axis_index("c")
    sid = lax.axis_index("s")
    wid = sid * _NC + core
    row0 = sid * _RPT
    ebase = wid * _EPT

    sb = (sb0, sb1, sb2, sb3)
    db = (db0, db1, db2, db3)
    rb = (rb0, rb1)
    semg = (semg0, semg1)
    semi = (semi0, semi1, semi2, semi3)
    sems = (sems0, sems1)

    _zero_acc(zb, acc_sh, row0, semz)
    plsc.subcore_barrier()

    def idx_start(c, k):
        off = ebase + c * _K
        pltpu.async_copy(src_hbm.at[pl.ds(off, _K)], sb[k % 4], semi[k % 4])
        pltpu.async_copy(dst_hbm.at[pl.ds(off, _K)], db[k % 4], semi[k % 4])

    def idx_wait(k):
        pltpu.make_async_copy(src_hbm.at[pl.ds(ebase, _K)],
                              sb[k % 4], semi[k % 4]).wait()
        pltpu.make_async_copy(dst_hbm.at[pl.ds(ebase, _K)],
                              db[k % 4], semi[k % 4]).wait()

    def gather_start(k):
        pltpu.async_copy(p_hbm.at[sb[k % 4]], rb[k % 2], semg[k % 2])

    def gather_wait(k):
        pltpu.make_async_copy(p_hbm.at[sb[k % 4]], rb[k % 2],
                              semg[k % 2]).wait()

    def scatter_start(k):
        pltpu.async_copy(rb[k % 2], acc_sh.at[db[k % 4]], sems[k % 2],
                         add=True)

    def scatter_wait(k):
        pltpu.make_async_copy(rb[k % 2], acc_sh.at[db[k % 4]],
                              sems[k % 2]).wait()

    idx_start(0, 0)
    idx_start(1, 1)

    def body(c, k, lead, last):
        if lead >= 2:
            scatter_wait(k - 2)
        idx_wait(k)
        gather_start(k)
        if lead >= 1:
            gather_wait(k - 1)
            scatter_start(k - 1)
        if not last:
            idx_start(c + 2, k + 2)

    for k in range(_UN):
        body(k, k, lead=k, last=False)

    @pl.loop(_UN, _NFULL - _UN, step=_UN)
    def _(j):
        for k in range(_UN):
            body(j + k, k, lead=2, last=False)

    base = _NFULL - _UN
    for k in range(_UN):
        body(base + k, k, lead=2, last=(base + k + 2 >= _NFULL))
    gather_wait(_UN - 1)
    scatter_start(_UN - 1)
    scatter_wait(_UN - 2)
    scatter_wait(_UN - 1)

    off2 = ebase + _NFULL * _K
    pltpu.sync_copy(src_hbm.at[pl.ds(off2, _REM)], sr)
    pltpu.sync_copy(dst_hbm.at[pl.ds(off2, _REM)], dr)
    pltpu.async_copy(p_hbm.at[sr], rb0.at[pl.ds(0, _REM)],
                     semg0).wait()
    pltpu.sync_copy(rb0.at[pl.ds(0, _REM)], acc_sh.at[dr], add=True)

    plsc.subcore_barrier()
    pltpu.sync_copy(acc_sh.at[pl.ds(row0, _RPT)],
                    agg_hbm.at[core].at[pl.ds(row0, _RPT)])


def _sc_segment_sum(p, src, dst):
    mesh = plsc.VectorSubcoreMesh(core_axis_name="c", subcore_axis_name="s",
                                  num_cores=_NC, num_subcores=_NS)
    fn = pl.kernel(
        _seg_body,
        out_type=jax.ShapeDtypeStruct((_NC, _NP, _D), jnp.float32),
        mesh=mesh,
        scratch_types=(
            pltpu.VMEM_SHARED((_NP, _D), jnp.float32),
            pltpu.VMEM((_K,), jnp.int32),
            pltpu.VMEM((_K,), jnp.int32),
            pltpu.VMEM((_K,), jnp.int32),
            pltpu.VMEM((_K,), jnp.int32),
            pltpu.VMEM((_K,), jnp.int32),
            pltpu.VMEM((_K,), jnp.int32),
            pltpu.VMEM((_K,), jnp.int32),
            pltpu.VMEM((_K,), jnp.int32),
            pltpu.VMEM((_K, _D), jnp.float32),
            pltpu.VMEM((_K, _D), jnp.float32),
            pltpu.VMEM((_REM,), jnp.int32),
            pltpu.VMEM((_REM,), jnp.int32),
            pltpu.VMEM((_ZR, _D), jnp.float32),
            pltpu.SemaphoreType.DMA,
            pltpu.SemaphoreType.DMA,
            pltpu.SemaphoreType.DMA,
            pltpu.SemaphoreType.DMA,
            pltpu.SemaphoreType.DMA,
            pltpu.SemaphoreType.DMA,
            pltpu.SemaphoreType.DMA,
            pltpu.SemaphoreType.DMA,
            pltpu.SemaphoreType.DMA,
        ))
    return fn(p, src, dst)


def _tc_counts(dst3):
    eb = 2000
    grid = (_E // eb,)

    def body(d_ref, out_ref, acc_s):
        i = pl.program_id(0)
        d = d_ref[0]
        hi = lax.broadcasted_iota(jnp.int32, (_HI, 1), 0)
        lo = lax.broadcasted_iota(jnp.int32, (_D, 1), 0)
        ohi = ((d >> 7) == hi).astype(jnp.bfloat16)
        olo = ((d & 127) == lo).astype(jnp.bfloat16)

        @pl.when(i == 0)
        def _():
            acc_s[...] = jnp.zeros_like(acc_s)

        acc_s[...] += lax.dot_general(
            ohi, olo, (((1,), (1,)), ((), ())),
            preferred_element_type=jnp.float32)

        @pl.when(i == _E // eb - 1)
        def _():
            out_ref[...] = acc_s[...]

    return pl.pallas_call(
        body,
        grid=grid,
        in_specs=[pl.BlockSpec((1, 1, eb), lambda i: (i, 0, 0))],
        out_specs=pl.BlockSpec((_HI, _D), lambda i: (0, 0)),
        out_shape=jax.ShapeDtypeStruct((_HI, _D), jnp.float32),
        scratch_shapes=[pltpu.VMEM((_HI, _D), jnp.float32)],
    )(dst3)


def _tc_matmul(x, w):
    rb = 2000
    grid = (_N // rb,)

    def body(x_ref, w_ref, o_ref):
        o_ref[...] = jnp.dot(x_ref[...], w_ref[...],
                             preferred_element_type=jnp.float32)

    return pl.pallas_call(
        body,
        grid=grid,
        in_specs=[pl.BlockSpec((rb, _D), lambda i: (i, 0)),
                  pl.BlockSpec((_D, _D), lambda i: (0, 0))],
        out_specs=pl.BlockSpec((rb, _D), lambda i: (i, 0)),
        out_shape=jax.ShapeDtypeStruct((_N, _D), jnp.float32),
    )(x, w)


def _tc_combine(l, agg, cnt2, bias, wr):
    rb = 2000
    grid = (_N // rb,)

    def body(l_ref, a_ref, c_ref, b_ref, wr_ref, h_ref, p_ref):
        a = a_ref[0] + a_ref[1]
        rinv = 1.0 / jnp.maximum(c_ref[...], 1.0)
        h = jnp.maximum(l_ref[...] + a * rinv + b_ref[...], 0.0)
        h_ref[...] = h
        p_ref[...] = jnp.dot(h, wr_ref[...],
                             preferred_element_type=jnp.float32)

    return pl.pallas_call(
        body,
        grid=grid,
        in_specs=[pl.BlockSpec((rb, _D), lambda i: (i, 0)),
                  pl.BlockSpec((_NC, rb, _D), lambda i: (0, i, 0)),
                  pl.BlockSpec((rb, 1), lambda i: (i, 0)),
                  pl.BlockSpec((1, _D), lambda i: (0, 0)),
                  pl.BlockSpec((_D, _D), lambda i: (0, 0))],
        out_specs=[pl.BlockSpec((rb, _D), lambda i: (i, 0)),
                   pl.BlockSpec((rb, _D), lambda i: (i, 0))],
        out_shape=[jax.ShapeDtypeStruct((_N, _D), jnp.float32)] * 2,
    )(l, agg, cnt2, bias, wr)


def _tc_pool_mlp(l2, agg2, cnt2, batch3, bias2, w1, bl1, w2, bl2):
    rb = 2000
    grid = (_N // rb,)

    def body(l_ref, a_ref, c_ref, bat_ref, b2_ref, w1_ref, bl1_ref,
             w2_ref, bl2_ref, out_ref, pooled_s, gcnt_s):
        i = pl.program_id(0)
        a = a_ref[0] + a_ref[1]
        rinv = 1.0 / jnp.maximum(c_ref[...], 1.0)
        h = jnp.maximum(l_ref[...] + a * rinv + b2_ref[...], 0.0)

        bat = bat_ref[0]
        seg = lax.broadcasted_iota(jnp.int32, (_B, 1), 0)
        onehot = (bat == seg).astype(jnp.float32)

        @pl.when(i == 0)
        def _():
            pooled_s[...] = jnp.zeros_like(pooled_s)
            gcnt_s[...] = jnp.zeros_like(gcnt_s)

        pooled_s[...] += jnp.dot(onehot, h,
                                 preferred_element_type=jnp.float32)
        gcnt_s[...] += jnp.sum(onehot, axis=1, keepdims=True)

        @pl.when(i == _N // rb - 1)
        def _():
            pm = pooled_s[...] / jnp.maximum(gcnt_s[...], 1.0)
            z = jnp.maximum(jnp.dot(pm, w1_ref[...],
                                    preferred_element_type=jnp.float32)
                            + bl1_ref[...], 0.0)
            out_ref[...] = jnp.dot(z, w2_ref[...],
                                   preferred_element_type=jnp.float32) \
                + bl2_ref[...]

    return pl.pallas_call(
        body,
        grid=grid,
        in_specs=[pl.BlockSpec((rb, _D), lambda i: (i, 0)),
                  pl.BlockSpec((_NC, rb, _D), lambda i: (0, i, 0)),
                  pl.BlockSpec((rb, 1), lambda i: (i, 0)),
                  pl.BlockSpec((1, 1, rb), lambda i: (i, 0, 0)),
                  pl.BlockSpec((1, _D), lambda i: (0, 0)),
                  pl.BlockSpec((_D, _D // 2), lambda i: (0, 0)),
                  pl.BlockSpec((1, _D // 2), lambda i: (0, 0)),
                  pl.BlockSpec((_D // 2, _D), lambda i: (0, 0)),
                  pl.BlockSpec((1, _D), lambda i: (0, 0))],
        out_specs=pl.BlockSpec((_B, _D), lambda i: (0, 0)),
        out_shape=jax.ShapeDtypeStruct((_B, _D), jnp.float32),
        scratch_shapes=[pltpu.VMEM((_B, _D), jnp.float32),
                        pltpu.VMEM((_B, 1), jnp.float32)],
    )(l2, agg2, cnt2, batch3, bias2, w1, bl1, w2, bl2)


def kernel(x, edge_index, batch, Wl1, Wr1, b1, Wl2, Wr2, b2,
           Wlin1, blin1, Wlin2, blin2):
    dst3 = edge_index[1].reshape(_E // 2000, 1, 2000)
    batch3 = batch.reshape(_N // 2000, 1, 2000)
    b1r = b1.reshape(1, _D)
    b2r = b2.reshape(1, _D)
    bl1r = blin1.reshape(1, _D // 2)
    bl2r = blin2.reshape(1, _D)

    src = edge_index[0]
    dst = edge_index[1]
    p1 = _tc_matmul(x, Wr1)
    agg1 = _sc_segment_sum(p1, src, dst)
    cnt2 = _tc_counts(dst3).reshape(_HI * _D, 1)
    l1 = _tc_matmul(x, Wl1)
    h1, p2 = _tc_combine(l1, agg1, cnt2, b1r, Wr2)
    agg2 = _sc_segment_sum(p2, src, dst)
    l2 = _tc_matmul(h1, Wl2)
    return _tc_pool_mlp(l2, agg2, cnt2, batch3, b2r, Wlin1, bl1r, Wlin2, bl2r)


# speedup vs baseline: 1.0033x; 1.0033x over previous
import jax
import jax.numpy as jnp
from jax import lax
from jax.experimental import pallas as pl
from jax.experimental.pallas import tpu as pltpu
from jax.experimental.pallas import tpu_sc as plsc

_N = 10000
_E = 320000
_D = 128
_B = 64

_NC = 2
_NS = 16
_NW = _NC * _NS
_EPT = _E // _NW
_K = 104
_NFULL = _EPT // _K
_REM = _EPT - _NFULL * _K
_NP = 10112
_RPT = _NP // _NS
_ZR = 32
_UN = 4

_HI = 80


def _zero_acc(zb, acc_sh, row0, semz):
    @pl.loop(0, _ZR)
    def _(r):
        @pl.loop(0, _D, step=16)
        def _(cc):
            zb[r, pl.ds(cc, 16)] = jnp.zeros((16,), jnp.float32)

    nfull = _RPT // _ZR
    rem = _RPT - nfull * _ZR
    for i in range(nfull):
        pltpu.async_copy(zb, acc_sh.at[pl.ds(row0 + i * _ZR, _ZR)], semz)
    pltpu.async_copy(zb.at[pl.ds(0, rem)],
                     acc_sh.at[pl.ds(row0 + nfull * _ZR, rem)], semz)
    for i in range(nfull):
        pltpu.make_async_copy(zb, acc_sh.at[pl.ds(row0 + i * _ZR, _ZR)],
                              semz).wait()
    pltpu.make_async_copy(zb.at[pl.ds(0, rem)],
                          acc_sh.at[pl.ds(row0 + nfull * _ZR, rem)],
                          semz).wait()


def _seg_body(p_hbm, src_hbm, dst_hbm, agg_hbm, acc_sh,
              sb0, sb1, sb2, sb3, db0, db1, db2, db3, rb0, rb1, sr, dr, zb,
              semg0, semg1, semi0, semi1, semi2, semi3, sems0, sems1, semz):
    core = lax.axis_index("c")
    sid = lax.axis_index("s")
    wid = sid * _NC + core
    row0 = sid * _RPT
    ebase = wid * _EPT

    sb = (sb0, sb1, sb2, sb3)
    db = (db0, db1, db2, db3)
    rb = (rb0, rb1)
    semg = (semg0, semg1)
    semi = (semi0, semi1, semi2, semi3)
    sems = (sems0, sems1)

    _zero_acc(zb, acc_sh, row0, semz)
    plsc.subcore_barrier()

    def idx_start(c, k):
        off = ebase + c * _K
        pltpu.async_copy(src_hbm.at[pl.ds(off, _K)], sb[k % 4], semi[k % 4])
        pltpu.async_copy(dst_hbm.at[pl.ds(off, _K)], db[k % 4], semi[k % 4])

    def idx_wait(k):
        pltpu.make_async_copy(src_hbm.at[pl.ds(ebase, _K)],
                              sb[k % 4], semi[k % 4]).wait()
        pltpu.make_async_copy(dst_hbm.at[pl.ds(ebase, _K)],
                              db[k % 4], semi[k % 4]).wait()

    def gather_start(k):
        pltpu.async_copy(p_hbm.at[sb[k % 4]], rb[k % 2], semg[k % 2])

    def gather_wait(k):
        pltpu.make_async_copy(p_hbm.at[sb[k % 4]], rb[k % 2],
                              semg[k % 2]).wait()

    def scatter_start(k):
        pltpu.async_copy(rb[k % 2], acc_sh.at[db[k % 4]], sems[k % 2],
                         add=True)

    def scatter_wait(k):
        pltpu.make_async_copy(rb[k % 2], acc_sh.at[db[k % 4]],
                              sems[k % 2]).wait()

    idx_start(0, 0)
    idx_start(1, 1)

    def body(c, k, lead, last):
        if lead >= 2:
            scatter_wait(k - 2)
        idx_wait(k)
        gather_start(k)
        if lead >= 1:
            gather_wait(k - 1)
            scatter_start(k - 1)
        if not last:
            idx_start(c + 2, k + 2)

    for k in range(_UN):
        body(k, k, lead=k, last=False)

    @pl.loop(_UN, _NFULL - _UN, step=_UN)
    def _(j):
        for k in range(_UN):
            body(j + k, k, lead=2, last=False)

    base = _NFULL - _UN
    for k in range(_UN):
        body(base + k, k, lead=2, last=(base + k + 2 >= _NFULL))
    gather_wait(_UN - 1)
    scatter_start(_UN - 1)
    scatter_wait(_UN - 2)
    scatter_wait(_UN - 1)

    off2 = ebase + _NFULL * _K
    pltpu.sync_copy(src_hbm.at[pl.ds(off2, _REM)], sr)
    pltpu.sync_copy(dst_hbm.at[pl.ds(off2, _REM)], dr)
    pltpu.async_copy(p_hbm.at[sr], rb0.at[pl.ds(0, _REM)],
                     semg0).wait()
    pltpu.sync_copy(rb0.at[pl.ds(0, _REM)], acc_sh.at[dr], add=True)

    plsc.subcore_barrier()
    pltpu.sync_copy(acc_sh.at[pl.ds(row0, _RPT)],
                    agg_hbm.at[core].at[pl.ds(row0, _RPT)])


def _sc_segment_sum(p, src, dst):
    mesh = plsc.VectorSubcoreMesh(core_axis_name="c", subcore_axis_name="s",
                                  num_cores=_NC, num_subcores=_NS)
    fn = pl.kernel(
        _seg_body,
        out_type=jax.ShapeDtypeStruct((_NC, _NP, _D), jnp.float32),
        mesh=mesh,
        scratch_types=(
            pltpu.VMEM_SHARED((_NP, _D), jnp.float32),
            pltpu.VMEM((_K,), jnp.int32),
            pltpu.VMEM((_K,), jnp.int32),
            pltpu.VMEM((_K,), jnp.int32),
            pltpu.VMEM((_K,), jnp.int32),
            pltpu.VMEM((_K,), jnp.int32),
            pltpu.VMEM((_K,), jnp.int32),
            pltpu.VMEM((_K,), jnp.int32),
            pltpu.VMEM((_K,), jnp.int32),
            pltpu.VMEM((_K, _D), jnp.float32),
            pltpu.VMEM((_K, _D), jnp.float32),
            pltpu.VMEM((_REM,), jnp.int32),
            pltpu.VMEM((_REM,), jnp.int32),
            pltpu.VMEM((_ZR, _D), jnp.float32),
            pltpu.SemaphoreType.DMA,
            pltpu.SemaphoreType.DMA,
            pltpu.SemaphoreType.DMA,
            pltpu.SemaphoreType.DMA,
            pltpu.SemaphoreType.DMA,
            pltpu.SemaphoreType.DMA,
            pltpu.SemaphoreType.DMA,
            pltpu.SemaphoreType.DMA,
            pltpu.SemaphoreType.DMA,
        ))
    return fn(p, src, dst)


def _tc_counts(dst3):
    eb = 2000
    grid = (_E // eb,)

    def body(d_ref, out_ref, acc_s):
        i = pl.program_id(0)
        d = d_ref[0]
        hi = lax.broadcasted_iota(jnp.int32, (_HI, 1), 0)
        lo = lax.broadcasted_iota(jnp.int32, (_D, 1), 0)
        ohi = ((d >> 7) == hi).astype(jnp.bfloat16)
        olo = ((d & 127) == lo).astype(jnp.bfloat16)

        @pl.when(i == 0)
        def _():
            acc_s[...] = jnp.zeros_like(acc_s)

        acc_s[...] += lax.dot_general(
            ohi, olo, (((1,), (1,)), ((), ())),
            preferred_element_type=jnp.float32)

        @pl.when(i == _E // eb - 1)
        def _():
            out_ref[...] = acc_s[...]

    return pl.pallas_call(
        body,
        grid=grid,
        in_specs=[pl.BlockSpec((1, 1, eb), lambda i: (i, 0, 0))],
        out_specs=pl.BlockSpec((_HI, _D), lambda i: (0, 0)),
        out_shape=jax.ShapeDtypeStruct((_HI, _D), jnp.float32),
        scratch_shapes=[pltpu.VMEM((_HI, _D), jnp.float32)],
    )(dst3)


def _tc_two_matmuls(x, wa, wb):
    rb = 2000
    grid = (_N // rb,)

    def body(x_ref, wa_ref, wb_ref, oa_ref, ob_ref):
        xb = x_ref[...]
        oa_ref[...] = jnp.dot(xb, wa_ref[...],
                              preferred_element_type=jnp.float32)
        ob_ref[...] = jnp.dot(xb, wb_ref[...],
                              preferred_element_type=jnp.float32)

    return pl.pallas_call(
        body,
        grid=grid,
        in_specs=[pl.BlockSpec((rb, _D), lambda i: (i, 0)),
                  pl.BlockSpec((_D, _D), lambda i: (0, 0)),
                  pl.BlockSpec((_D, _D), lambda i: (0, 0))],
        out_specs=[pl.BlockSpec((rb, _D), lambda i: (i, 0)),
                   pl.BlockSpec((rb, _D), lambda i: (i, 0))],
        out_shape=[jax.ShapeDtypeStruct((_N, _D), jnp.float32)] * 2,
    )(x, wa, wb)


def _tc_combine(l, agg, cnt2, bias, wr, wl):
    rb = 2000
    grid = (_N // rb,)

    def body(l_ref, a_ref, c_ref, b_ref, wr_ref, wl_ref, p_ref, o_ref):
        a = a_ref[0] + a_ref[1]
        rinv = 1.0 / jnp.maximum(c_ref[...], 1.0)
        h = jnp.maximum(l_ref[...] + a * rinv + b_ref[...], 0.0)
        p_ref[...] = jnp.dot(h, wr_ref[...],
                             preferred_element_type=jnp.float32)
        o_ref[...] = jnp.dot(h, wl_ref[...],
                             preferred_element_type=jnp.float32)

    return pl.pallas_call(
        body,
        grid=grid,
        in_specs=[pl.BlockSpec((rb, _D), lambda i: (i, 0)),
                  pl.BlockSpec((_NC, rb, _D), lambda i: (0, i, 0)),
                  pl.BlockSpec((rb, 1), lambda i: (i, 0)),
                  pl.BlockSpec((1, _D), lambda i: (0, 0)),
                  pl.BlockSpec((_D, _D), lambda i: (0, 0)),
                  pl.BlockSpec((_D, _D), lambda i: (0, 0))],
        out_specs=[pl.BlockSpec((rb, _D), lambda i: (i, 0)),
                   pl.BlockSpec((rb, _D), lambda i: (i, 0))],
        out_shape=[jax.ShapeDtypeStruct((_N, _D), jnp.float32)] * 2,
    )(l, agg, cnt2, bias, wr, wl)


def _tc_pool_mlp(l2, agg2, cnt2, batch3, bias2, w1, bl1, w2, bl2):
    rb = 2000
    grid = (_N // rb,)

    def body(l_ref, a_ref, c_ref, bat_ref, b2_ref, w1_ref, bl1_ref,
             w2_ref, bl2_ref, out_ref, pooled_s, gcnt_s):
        i = pl.program_id(0)
        a = a_ref[0] + a_ref[1]
        rinv = 1.0 / jnp.maximum(c_ref[...], 1.0)
        h = jnp.maximum(l_ref[...] + a * rinv + b2_ref[...], 0.0)

        bat = bat_ref[0]
        seg = lax.broadcasted_iota(jnp.int32, (_B, 1), 0)
        onehot = (bat == seg).astype(jnp.float32)

        @pl.when(i == 0)
        def _():
            pooled_s[...] = jnp.zeros_like(pooled_s)
            gcnt_s[...] = jnp.zeros_like(gcnt_s)

        pooled_s[...] += jnp.dot(onehot, h,
                                 preferred_element_type=jnp.float32)
        gcnt_s[...] += jnp.sum(onehot, axis=1, keepdims=True)

        @pl.when(i == _N // rb - 1)
        def _():
            pm = pooled_s[...] / jnp.maximum(gcnt_s[...], 1.0)
            z = jnp.maximum(jnp.dot(pm, w1_ref[...],
                                    preferred_element_type=jnp.float32)
                            + bl1_ref[...], 0.0)
            out_ref[...] = jnp.dot(z, w2_ref[...],
                                   preferred_element_type=jnp.float32) \
                + bl2_ref[...]

    return pl.pallas_call(
        body,
        grid=grid,
        in_specs=[pl.BlockSpec((rb, _D), lambda i: (i, 0)),
                  pl.BlockSpec((_NC, rb, _D), lambda i: (0, i, 0)),
                  pl.BlockSpec((rb, 1), lambda i: (i, 0)),
                  pl.BlockSpec((1, 1, rb), lambda i: (i, 0, 0)),
                  pl.BlockSpec((1, _D), lambda i: (0, 0)),
                  pl.BlockSpec((_D, _D // 2), lambda i: (0, 0)),
                  pl.BlockSpec((1, _D // 2), lambda i: (0, 0)),
                  pl.BlockSpec((_D // 2, _D), lambda i: (0, 0)),
                  pl.BlockSpec((1, _D), lambda i: (0, 0))],
        out_specs=pl.BlockSpec((_B, _D), lambda i: (0, 0)),
        out_shape=jax.ShapeDtypeStruct((_B, _D), jnp.float32),
        scratch_shapes=[pltpu.VMEM((_B, _D), jnp.float32),
                        pltpu.VMEM((_B, 1), jnp.float32)],
    )(l2, agg2, cnt2, batch3, bias2, w1, bl1, w2, bl2)


def kernel(x, edge_index, batch, Wl1, Wr1, b1, Wl2, Wr2, b2,
           Wlin1, blin1, Wlin2, blin2):
    dst3 = edge_index[1].reshape(_E // 2000, 1, 2000)
    batch3 = batch.reshape(_N // 2000, 1, 2000)
    b1r = b1.reshape(1, _D)
    b2r = b2.reshape(1, _D)
    bl1r = blin1.reshape(1, _D // 2)
    bl2r = blin2.reshape(1, _D)

    src = edge_index[0]
    dst = edge_index[1]
    cnt2 = _tc_counts(dst3).reshape(_HI * _D, 1)
    p1, l1 = _tc_two_matmuls(x, Wr1, Wl1)
    agg1 = _sc_segment_sum(p1, src, dst)
    p2, l2 = _tc_combine(l1, agg1, cnt2, b1r, Wr2, Wl2)
    agg2 = _sc_segment_sum(p2, src, dst)
    return _tc_pool_mlp(l2, agg2, cnt2, batch3, b2r, Wlin1, bl1r, Wlin2, bl2r)


# speedup vs baseline: 1.0286x; 1.0252x over previous
import jax
import jax.numpy as jnp
from jax import lax
from jax.experimental import pallas as pl
from jax.experimental.pallas import tpu as pltpu
from jax.experimental.pallas import tpu_sc as plsc

_N = 10000
_E = 320000
_D = 128
_B = 64

_NC = 2
_NS = 16
_NW = _NC * _NS
_EPT = _E // _NW
_K = 128
_NFULL = _EPT // _K
_REM = _EPT - _NFULL * _K
_NP = 10112
_RPT = _NP // _NS
_ZR = 8
_UN = 4

_HI = 80


def _zero_acc(zb, acc_sh, row0, semz):
    @pl.loop(0, _ZR)
    def _(r):
        @pl.loop(0, _D, step=16)
        def _(cc):
            zb[r, pl.ds(cc, 16)] = jnp.zeros((16,), jnp.float32)

    nfull = _RPT // _ZR
    for i in range(nfull):
        pltpu.async_copy(zb, acc_sh.at[pl.ds(row0 + i * _ZR, _ZR)], semz)
    for i in range(nfull):
        pltpu.make_async_copy(zb, acc_sh.at[pl.ds(row0 + i * _ZR, _ZR)],
                              semz).wait()


def _seg_body(p_hbm, src_hbm, dst_hbm, agg_hbm, acc_sh,
              sb0, sb1, sb2, sb3, db0, db1, db2, db3, rb0, rb1, sr, dr, zb,
              semg0, semg1, semi0, semi1, semi2, semi3, sems0, sems1, semz):
    core = lax.axis_index("c")
    sid = lax.axis_index("s")
    wid = sid * _NC + core
    row0 = sid * _RPT
    ebase = wid * _EPT

    sb = (sb0, sb1, sb2, sb3)
    db = (db0, db1, db2, db3)
    rb = (rb0, rb1)
    semg = (semg0, semg1)
    semi = (semi0, semi1, semi2, semi3)
    sems = (sems0, sems1)

    _zero_acc(zb, acc_sh, row0, semz)
    plsc.subcore_barrier()

    def idx_start(c, k):
        off = ebase + c * _K
        pltpu.async_copy(src_hbm.at[pl.ds(off, _K)], sb[k % 4], semi[k % 4])
        pltpu.async_copy(dst_hbm.at[pl.ds(off, _K)], db[k % 4], semi[k % 4])

    def idx_wait(k):
        pltpu.make_async_copy(src_hbm.at[pl.ds(ebase, _K)],
                              sb[k % 4], semi[k % 4]).wait()
        pltpu.make_async_copy(dst_hbm.at[pl.ds(ebase, _K)],
                              db[k % 4], semi[k % 4]).wait()

    def gather_start(k):
        pltpu.async_copy(p_hbm.at[sb[k % 4]], rb[k % 2], semg[k % 2])

    def gather_wait(k):
        pltpu.make_async_copy(p_hbm.at[sb[k % 4]], rb[k % 2],
                              semg[k % 2]).wait()

    def scatter_start(k):
        pltpu.async_copy(rb[k % 2], acc_sh.at[db[k % 4]], sems[k % 2],
                         add=True)

    def scatter_wait(k):
        pltpu.make_async_copy(rb[k % 2], acc_sh.at[db[k % 4]],
                              sems[k % 2]).wait()

    idx_start(0, 0)
    idx_start(1, 1)

    def body(c, k, lead, last):
        if lead >= 2:
            scatter_wait(k - 2)
        idx_wait(k)
        gather_start(k)
        if lead >= 1:
            gather_wait(k - 1)
            scatter_start(k - 1)
        if not last:
            idx_start(c + 2, k + 2)

    _TAIL = _UN + (_NFULL - 2 * _UN) % _UN
    for k in range(_UN):
        body(k, k, lead=k, last=False)

    @pl.loop(_UN, _NFULL - _TAIL, step=_UN)
    def _(j):
        for k in range(_UN):
            body(j + k, k, lead=2, last=False)

    base = _NFULL - _TAIL
    for t in range(_TAIL):
        body(base + t, t, lead=2, last=(base + t + 2 >= _NFULL))
    gather_wait(_TAIL - 1)
    scatter_start(_TAIL - 1)
    scatter_wait(_TAIL - 2)
    scatter_wait(_TAIL - 1)

    off2 = ebase + _NFULL * _K
    pltpu.sync_copy(src_hbm.at[pl.ds(off2, _REM)], sr)
    pltpu.sync_copy(dst_hbm.at[pl.ds(off2, _REM)], dr)
    pltpu.async_copy(p_hbm.at[sr], rb0.at[pl.ds(0, _REM)],
                     semg0).wait()
    pltpu.sync_copy(rb0.at[pl.ds(0, _REM)], acc_sh.at[dr], add=True)

    plsc.subcore_barrier()
    pltpu.sync_copy(acc_sh.at[pl.ds(row0, _RPT)],
                    agg_hbm.at[core].at[pl.ds(row0, _RPT)])


def _sc_segment_sum(p, src, dst):
    mesh = plsc.VectorSubcoreMesh(core_axis_name="c", subcore_axis_name="s",
                                  num_cores=_NC, num_subcores=_NS)
    fn = pl.kernel(
        _seg_body,
        out_type=jax.ShapeDtypeStruct((_NC, _NP, _D), jnp.float32),
        mesh=mesh,
        scratch_types=(
            pltpu.VMEM_SHARED((_NP, _D), jnp.float32),
            pltpu.VMEM((_K,), jnp.int32),
            pltpu.VMEM((_K,), jnp.int32),
            pltpu.VMEM((_K,), jnp.int32),
            pltpu.VMEM((_K,), jnp.int32),
            pltpu.VMEM((_K,), jnp.int32),
            pltpu.VMEM((_K,), jnp.int32),
            pltpu.VMEM((_K,), jnp.int32),
            pltpu.VMEM((_K,), jnp.int32),
            pltpu.VMEM((_K, _D), jnp.float32),
            pltpu.VMEM((_K, _D), jnp.float32),
            pltpu.VMEM((_REM,), jnp.int32),
            pltpu.VMEM((_REM,), jnp.int32),
            pltpu.VMEM((_ZR, _D), jnp.float32),
            pltpu.SemaphoreType.DMA,
            pltpu.SemaphoreType.DMA,
            pltpu.SemaphoreType.DMA,
            pltpu.SemaphoreType.DMA,
            pltpu.SemaphoreType.DMA,
            pltpu.SemaphoreType.DMA,
            pltpu.SemaphoreType.DMA,
            pltpu.SemaphoreType.DMA,
            pltpu.SemaphoreType.DMA,
        ))
    return fn(p, src, dst)


def _tc_counts(dst3):
    eb = 2000
    grid = (_E // eb,)

    def body(d_ref, out_ref, acc_s):
        i = pl.program_id(0)
        d = d_ref[0]
        hi = lax.broadcasted_iota(jnp.int32, (_HI, 1), 0)
        lo = lax.broadcasted_iota(jnp.int32, (_D, 1), 0)
        ohi = ((d >> 7) == hi).astype(jnp.bfloat16)
        olo = ((d & 127) == lo).astype(jnp.bfloat16)

        @pl.when(i == 0)
        def _():
            acc_s[...] = jnp.zeros_like(acc_s)

        acc_s[...] += lax.dot_general(
            ohi, olo, (((1,), (1,)), ((), ())),
            preferred_element_type=jnp.float32)

        @pl.when(i == _E // eb - 1)
        def _():
            out_ref[...] = acc_s[...]

    return pl.pallas_call(
        body,
        grid=grid,
        in_specs=[pl.BlockSpec((1, 1, eb), lambda i: (i, 0, 0))],
        out_specs=pl.BlockSpec((_HI, _D), lambda i: (0, 0)),
        out_shape=jax.ShapeDtypeStruct((_HI, _D), jnp.float32),
        scratch_shapes=[pltpu.VMEM((_HI, _D), jnp.float32)],
    )(dst3)


def _tc_two_matmuls(x, wa, wb):
    rb = 2000
    grid = (_N // rb,)

    def body(x_ref, wa_ref, wb_ref, oa_ref, ob_ref):
        xb = x_ref[...]
        oa_ref[...] = jnp.dot(xb, wa_ref[...],
                              preferred_element_type=jnp.float32)
        ob_ref[...] = jnp.dot(xb, wb_ref[...],
                              preferred_element_type=jnp.float32)

    return pl.pallas_call(
        body,
        grid=grid,
        in_specs=[pl.BlockSpec((rb, _D), lambda i: (i, 0)),
                  pl.BlockSpec((_D, _D), lambda i: (0, 0)),
                  pl.BlockSpec((_D, _D), lambda i: (0, 0))],
        out_specs=[pl.BlockSpec((rb, _D), lambda i: (i, 0)),
                   pl.BlockSpec((rb, _D), lambda i: (i, 0))],
        out_shape=[jax.ShapeDtypeStruct((_N, _D), jnp.float32)] * 2,
    )(x, wa, wb)


def _tc_combine(l, agg, cnt2, bias, wr, wl):
    rb = 2000
    grid = (_N // rb,)

    def body(l_ref, a_ref, c_ref, b_ref, wr_ref, wl_ref, p_ref, o_ref):
        a = a_ref[0] + a_ref[1]
        rinv = 1.0 / jnp.maximum(c_ref[...], 1.0)
        h = jnp.maximum(l_ref[...] + a * rinv + b_ref[...], 0.0)
        p_ref[...] = jnp.dot(h, wr_ref[...],
                             preferred_element_type=jnp.float32)
        o_ref[...] = jnp.dot(h, wl_ref[...],
                             preferred_element_type=jnp.float32)

    return pl.pallas_call(
        body,
        grid=grid,
        in_specs=[pl.BlockSpec((rb, _D), lambda i: (i, 0)),
                  pl.BlockSpec((_NC, rb, _D), lambda i: (0, i, 0)),
                  pl.BlockSpec((rb, 1), lambda i: (i, 0)),
                  pl.BlockSpec((1, _D), lambda i: (0, 0)),
                  pl.BlockSpec((_D, _D), lambda i: (0, 0)),
                  pl.BlockSpec((_D, _D), lambda i: (0, 0))],
        out_specs=[pl.BlockSpec((rb, _D), lambda i: (i, 0)),
                   pl.BlockSpec((rb, _D), lambda i: (i, 0))],
        out_shape=[jax.ShapeDtypeStruct((_N, _D), jnp.float32)] * 2,
    )(l, agg, cnt2, bias, wr, wl)


def _tc_pool_mlp(l2, agg2, cnt2, batch3, bias2, w1, bl1, w2, bl2):
    rb = 2000
    grid = (_N // rb,)

    def body(l_ref, a_ref, c_ref, bat_ref, b2_ref, w1_ref, bl1_ref,
             w2_ref, bl2_ref, out_ref, pooled_s, gcnt_s):
        i = pl.program_id(0)
        a = a_ref[0] + a_ref[1]
        rinv = 1.0 / jnp.maximum(c_ref[...], 1.0)
        h = jnp.maximum(l_ref[...] + a * rinv + b2_ref[...], 0.0)

        bat = bat_ref[0]
        seg = lax.broadcasted_iota(jnp.int32, (_B, 1), 0)
        onehot = (bat == seg).astype(jnp.float32)

        @pl.when(i == 0)
        def _():
            pooled_s[...] = jnp.zeros_like(pooled_s)
            gcnt_s[...] = jnp.zeros_like(gcnt_s)

        pooled_s[...] += jnp.dot(onehot, h,
                                 preferred_element_type=jnp.float32)
        gcnt_s[...] += jnp.sum(onehot, axis=1, keepdims=True)

        @pl.when(i == _N // rb - 1)
        def _():
            pm = pooled_s[...] / jnp.maximum(gcnt_s[...], 1.0)
            z = jnp.maximum(jnp.dot(pm, w1_ref[...],
                                    preferred_element_type=jnp.float32)
                            + bl1_ref[...], 0.0)
            out_ref[...] = jnp.dot(z, w2_ref[...],
                                   preferred_element_type=jnp.float32) \
                + bl2_ref[...]

    return pl.pallas_call(
        body,
        grid=grid,
        in_specs=[pl.BlockSpec((rb, _D), lambda i: (i, 0)),
                  pl.BlockSpec((_NC, rb, _D), lambda i: (0, i, 0)),
                  pl.BlockSpec((rb, 1), lambda i: (i, 0)),
                  pl.BlockSpec((1, 1, rb), lambda i: (i, 0, 0)),
                  pl.BlockSpec((1, _D), lambda i: (0, 0)),
                  pl.BlockSpec((_D, _D // 2), lambda i: (0, 0)),
                  pl.BlockSpec((1, _D // 2), lambda i: (0, 0)),
                  pl.BlockSpec((_D // 2, _D), lambda i: (0, 0)),
                  pl.BlockSpec((1, _D), lambda i: (0, 0))],
        out_specs=pl.BlockSpec((_B, _D), lambda i: (0, 0)),
        out_shape=jax.ShapeDtypeStruct((_B, _D), jnp.float32),
        scratch_shapes=[pltpu.VMEM((_B, _D), jnp.float32),
                        pltpu.VMEM((_B, 1), jnp.float32)],
    )(l2, agg2, cnt2, batch3, bias2, w1, bl1, w2, bl2)


def kernel(x, edge_index, batch, Wl1, Wr1, b1, Wl2, Wr2, b2,
           Wlin1, blin1, Wlin2, blin2):
    dst3 = edge_index[1].reshape(_E // 2000, 1, 2000)
    batch3 = batch.reshape(_N // 2000, 1, 2000)
    b1r = b1.reshape(1, _D)
    b2r = b2.reshape(1, _D)
    bl1r = blin1.reshape(1, _D // 2)
    bl2r = blin2.reshape(1, _D)

    src = edge_index[0]
    dst = edge_index[1]
    cnt2 = _tc_counts(dst3).reshape(_HI * _D, 1)
    p1, l1 = _tc_two_matmuls(x, Wr1, Wl1)
    agg1 = _sc_segment_sum(p1, src, dst)
    p2, l2 = _tc_combine(l1, agg1, cnt2, b1r, Wr2, Wl2)
    agg2 = _sc_segment_sum(p2, src, dst)
    return _tc_pool_mlp(l2, agg2, cnt2, batch3, b2r, Wlin1, bl1r, Wlin2, bl2r)
